# Initial kernel scaffold; baseline (speedup 1.0000x reference)
import jax
import jax.numpy as jnp
from jax.experimental import pallas as pl


def kernel(context_ids, doc_ids, target_noise_ids, D, W, O):
    raise NotImplementedError("write your pallas kernel here")


# speedup vs baseline: 2.3509x; 2.3509x over previous
import functools

import jax
import jax.numpy as jnp
from jax import lax
from jax.experimental import pallas as pl
from jax.experimental.pallas import tpu as pltpu
from jax.experimental.pallas import tpu_sc as plsc

VEC = 128
BATCH = 4096
CTX = 20
NOISE = 26
OPAD = 32

NC = 2
NS = 16
NW = NC * NS
BPW = BATCH // NW
SUB = 4
NIT = BPW // SUB
CCH = SUB * CTX
TCH = SUB * NOISE
NLG = VEC // 16


def _lane_perm(v, idx):
    dnums = lax.GatherDimensionNumbers(
        offset_dims=(), collapsed_slice_dims=(0,), start_index_map=(0,))
    return lax.gather(v, idx[:, None], dnums, (1,),
                      mode=lax.GatherScatterMode.PROMISE_IN_BOUNDS)


def _tr_body(o_ref, ot_ref):
    ot_ref[...] = o_ref[...].T


def _transpose(o):
    n = o.shape[1]
    blk = 2048
    return pl.pallas_call(
        _tr_body,
        grid=(pl.cdiv(n, blk),),
        in_specs=[pl.BlockSpec((VEC, blk), lambda i: (0, i))],
        out_specs=pl.BlockSpec((blk, VEC), lambda i: (i, 0)),
        out_shape=jax.ShapeDtypeStruct((n, VEC), jnp.float32),
    )(o)


def _sc_body(doc_hbm, ctx_hbm, tn_hbm, d_hbm, w_hbm, ot_hbm, out_hbm,
             doc_idx, ctx_idx, tn_idx, docbuf, xbuf, cbuf, obuf,
             outv, sem):
    c = lax.axis_index("c")
    s = lax.axis_index("s")
    wid = s * NC + c

    pltpu.sync_copy(doc_hbm.at[wid], doc_idx)
    pltpu.sync_copy(ctx_hbm.at[wid], ctx_idx)
    pltpu.sync_copy(tn_hbm.at[wid], tn_idx)

    pltpu.async_copy(d_hbm.at[doc_idx], docbuf, sem).wait()

    def p1(i, carry):
        pltpu.async_copy(w_hbm.at[ctx_idx.at[i]], cbuf, sem).wait()
        for bb in range(SUB):
            b = i * SUB + bb
            for v in range(NLG):
                sl = pl.ds(v * 16, 16)
                acc = docbuf[b, sl]
                for j in range(CTX):
                    acc = acc + cbuf[bb * CTX + j, sl]
                xbuf[b, sl] = acc
        return carry

    lax.fori_loop(0, NIT, p1, 0)

    lanes = lax.iota(jnp.int32, 16)
    perms = [lanes ^ sh for sh in (8, 4, 2, 1)]
    masks = [lanes == (t % 16) for t in range(NOISE)]
    zeros16 = jnp.zeros((16,), jnp.float32)

    def p2(i, carry):
        pltpu.async_copy(ot_hbm.at[tn_idx.at[i]], obuf, sem).wait()
        for bb in range(SUB):
            b = i * SUB + bb
            xv = [xbuf[b, pl.ds(v * 16, 16)] for v in range(NLG)]
            og = [zeros16, zeros16]
            for t in range(NOISE):
                r = bb * NOISE + t
                acc = xv[0] * obuf[r, pl.ds(0, 16)]
                for v in range(1, NLG):
                    acc = acc + xv[v] * obuf[r, pl.ds(v * 16, 16)]
                for p in perms:
                    acc = acc + _lane_perm(acc, p)
                og[t // 16] = jnp.where(masks[t], acc, og[t // 16])
            outv[bb, pl.ds(0, 16)] = og[0]
            outv[bb, pl.ds(16, 16)] = og[1]
        pltpu.sync_copy(outv, out_hbm.at[pl.ds(wid * BPW + i * SUB, SUB)])
        return carry

    lax.fori_loop(0, NIT, p2, 0)


@functools.partial(jax.jit)
def _sc_fwd(doc, ctx, tn, d, w, ot):
    mesh = plsc.VectorSubcoreMesh(core_axis_name="c", subcore_axis_name="s")
    run = pl.kernel(
        _sc_body,
        mesh=mesh,
        out_type=jax.ShapeDtypeStruct((BATCH, OPAD), jnp.float32),
        scratch_types=[
            pltpu.VMEM((BPW,), jnp.int32),
            pltpu.VMEM((NIT, CCH), jnp.int32),
            pltpu.VMEM((NIT, TCH), jnp.int32),
            pltpu.VMEM((BPW, VEC), jnp.float32),
            pltpu.VMEM((BPW, VEC), jnp.float32),
            pltpu.VMEM((CCH, VEC), jnp.float32),
            pltpu.VMEM((TCH, VEC), jnp.float32),
            pltpu.VMEM((SUB, OPAD), jnp.float32),
            pltpu.SemaphoreType.DMA,
        ],
    )
    return run(doc, ctx, tn, d, w, ot)


def kernel(context_ids, doc_ids, target_noise_ids, D, W, O):
    ot = _transpose(O)
    doc = doc_ids.reshape(NW, BPW)
    ctx = context_ids.reshape(NW, NIT, CCH)
    tn = target_noise_ids.reshape(NW, NIT, TCH)
    out = _sc_fwd(doc, ctx, tn, D, W, ot)
    return out[:, :NOISE]


# speedup vs baseline: 3.8797x; 1.6503x over previous
import functools

import jax
import jax.numpy as jnp
from jax import lax
from jax.experimental import pallas as pl
from jax.experimental.pallas import tpu as pltpu
from jax.experimental.pallas import tpu_sc as plsc

VEC = 128
BATCH = 4096
CTX = 20
NOISE = 26
OPAD = 32

NC = 2
NS = 16
NW = NC * NS
BPW = BATCH // NW
SUB = 4
NIT = BPW // SUB
CCH = SUB * CTX
TCH = SUB * NOISE
NLG = VEC // 16


def _lane_perm(v, idx):
    dnums = lax.GatherDimensionNumbers(
        offset_dims=(), collapsed_slice_dims=(0,), start_index_map=(0,))
    return lax.gather(v, idx[:, None], dnums, (1,),
                      mode=lax.GatherScatterMode.PROMISE_IN_BOUNDS)


def _tree_sum(vals):
    while len(vals) > 1:
        nxt = [vals[k] + vals[k + 1] for k in range(0, len(vals) - 1, 2)]
        if len(vals) % 2:
            nxt.append(vals[-1])
        vals = nxt
    return vals[0]


def _sc_body(doc_hbm, ctx_hbm, tn_hbm, d_hbm, w_hbm, ot_hbm, out_hbm,
             doc_idx, ctx_idx, tn_idx, docbuf, xbuf, cb0, cb1, ob0, ob1,
             outv, sem_d, sem_c0, sem_c1, sem_o0, sem_o1):
    c = lax.axis_index("c")
    s = lax.axis_index("s")
    wid = s * NC + c

    pltpu.sync_copy(doc_hbm.at[wid], doc_idx)
    pltpu.sync_copy(ctx_hbm.at[wid], ctx_idx)
    pltpu.sync_copy(tn_hbm.at[wid], tn_idx)

    def ctx_dma(i, buf, sem):
        return pltpu.make_async_copy(w_hbm.at[ctx_idx.at[i]], buf, sem)

    def ot_dma(i, buf, sem):
        return pltpu.make_async_copy(ot_hbm.at[tn_idx.at[i]], buf, sem)

    doc_dma = pltpu.make_async_copy(d_hbm.at[doc_idx], docbuf, sem_d)
    doc_dma.start()
    ctx_dma(0, cb0, sem_c0).start()
    ctx_dma(1, cb1, sem_c1).start()
    ot_dma(0, ob0, sem_o0).start()
    ot_dma(1, ob1, sem_o1).start()
    doc_dma.wait()

    def seg_sum(i, buf):
        for bb in range(SUB):
            b = i * SUB + bb
            for v in range(NLG):
                sl = pl.ds(v * 16, 16)
                xbuf[b, sl] = _tree_sum(
                    [docbuf[b, sl]]
                    + [buf[bb * CTX + j, sl] for j in range(CTX)])

    def p1(k, carry):
        i0 = 2 * k
        ctx_dma(i0, cb0, sem_c0).wait()
        seg_sum(i0, cb0)
        ctx_dma(i0 + 2, cb0, sem_c0).start()
        ctx_dma(i0 + 1, cb1, sem_c1).wait()
        seg_sum(i0 + 1, cb1)
        ctx_dma(i0 + 3, cb1, sem_c1).start()
        return carry

    lax.fori_loop(0, NIT // 2 - 1, p1, 0)
    ctx_dma(NIT - 2, cb0, sem_c0).wait()
    seg_sum(NIT - 2, cb0)
    ctx_dma(NIT - 1, cb1, sem_c1).wait()
    seg_sum(NIT - 1, cb1)

    lanes = lax.iota(jnp.int32, 16)
    perms = [lanes ^ sh for sh in (8, 4, 2, 1)]
    masks = [lanes == (t % 16) for t in range(NOISE)]
    zeros16 = jnp.zeros((16,), jnp.float32)

    def dots(i, buf):
        for bb in range(SUB):
            b = i * SUB + bb
            xv = [xbuf[b, pl.ds(v * 16, 16)] for v in range(NLG)]
            og = [zeros16, zeros16]
            for t in range(NOISE):
                r = bb * NOISE + t
                acc = _tree_sum([xv[v] * buf[r, pl.ds(v * 16, 16)]
                                 for v in range(NLG)])
                for p in perms:
                    acc = acc + _lane_perm(acc, p)
                og[t // 16] = jnp.where(masks[t], acc, og[t // 16])
            outv[bb, pl.ds(0, 16)] = og[0]
            outv[bb, pl.ds(16, 16)] = og[1]
        pltpu.sync_copy(outv, out_hbm.at[pl.ds(wid * BPW + i * SUB, SUB)])

    def p2(k, carry):
        i0 = 2 * k
        ot_dma(i0, ob0, sem_o0).wait()
        dots(i0, ob0)
        ot_dma(i0 + 2, ob0, sem_o0).start()
        ot_dma(i0 + 1, ob1, sem_o1).wait()
        dots(i0 + 1, ob1)
        ot_dma(i0 + 3, ob1, sem_o1).start()
        return carry

    lax.fori_loop(0, NIT // 2 - 1, p2, 0)
    ot_dma(NIT - 2, ob0, sem_o0).wait()
    dots(NIT - 2, ob0)
    ot_dma(NIT - 1, ob1, sem_o1).wait()
    dots(NIT - 1, ob1)


@functools.partial(jax.jit)
def _sc_fwd(doc, ctx, tn, d, w, ot):
    mesh = plsc.VectorSubcoreMesh(core_axis_name="c", subcore_axis_name="s")
    run = pl.kernel(
        _sc_body,
        mesh=mesh,
        out_type=jax.ShapeDtypeStruct((BATCH, OPAD), jnp.float32),
        scratch_types=[
            pltpu.VMEM((BPW,), jnp.int32),
            pltpu.VMEM((NIT, CCH), jnp.int32),
            pltpu.VMEM((NIT, TCH), jnp.int32),
            pltpu.VMEM((BPW, VEC), jnp.float32),
            pltpu.VMEM((BPW, VEC), jnp.float32),
            pltpu.VMEM((CCH, VEC), jnp.float32),
            pltpu.VMEM((CCH, VEC), jnp.float32),
            pltpu.VMEM((TCH, VEC), jnp.float32),
            pltpu.VMEM((TCH, VEC), jnp.float32),
            pltpu.VMEM((SUB, OPAD), jnp.float32),
            pltpu.SemaphoreType.DMA,
            pltpu.SemaphoreType.DMA,
            pltpu.SemaphoreType.DMA,
            pltpu.SemaphoreType.DMA,
            pltpu.SemaphoreType.DMA,
        ],
    )
    return run(doc, ctx, tn, d, w, ot)


def kernel(context_ids, doc_ids, target_noise_ids, D, W, O):
    ot = jnp.swapaxes(O, 0, 1)
    doc = doc_ids.reshape(NW, BPW)
    ctx = context_ids.reshape(NW, NIT, CCH)
    tn = target_noise_ids.reshape(NW, NIT, TCH)
    out = _sc_fwd(doc, ctx, tn, D, W, ot)
    return out[:, :NOISE]


# speedup vs baseline: 7.4724x; 1.9261x over previous
import functools

import jax
import jax.numpy as jnp
from jax import lax
from jax.experimental import pallas as pl
from jax.experimental.pallas import tpu as pltpu
from jax.experimental.pallas import tpu_sc as plsc

VEC = 128
BATCH = 4096
CTX = 20
NOISE = 26
OPAD = 32

NC = 2
NS = 16
NW = NC * NS
BPW = BATCH // NW
SUB = 4
NIT = BPW // SUB
CCH = SUB * CTX
TCH = SUB * NOISE
NLG = VEC // 16
RING = 4


def _lane_perm(v, idx):
    dnums = lax.GatherDimensionNumbers(
        offset_dims=(), collapsed_slice_dims=(0,), start_index_map=(0,))
    return lax.gather(v, idx[:, None], dnums, (1,),
                      mode=lax.GatherScatterMode.PROMISE_IN_BOUNDS)


def _tree_sum(vals):
    while len(vals) > 1:
        nxt = [vals[k] + vals[k + 1] for k in range(0, len(vals) - 1, 2)]
        if len(vals) % 2:
            nxt.append(vals[-1])
        vals = nxt
    return vals[0]


def _sc_body(doc_hbm, ctx_hbm, tn_hbm, d_hbm, w_hbm, ot_hbm, out_hbm,
             doc_idx, ctx_idx, tn_idx, xbuf,
             gb0, gb1, gb2, gb3, outv,
             sem_d, sem_c0, sem_c1, sem_c2, sem_c3,
             sem_o0, sem_o1, sem_o2, sem_o3):
    c = lax.axis_index("c")
    s = lax.axis_index("s")
    wid = s * NC + c

    bufs = [gb0, gb1, gb2, gb3]
    sem_cs = [sem_c0, sem_c1, sem_c2, sem_c3]
    sem_os = [sem_o0, sem_o1, sem_o2, sem_o3]

    pltpu.sync_copy(doc_hbm.at[wid], doc_idx)
    pltpu.sync_copy(ctx_hbm.at[wid], ctx_idx)
    pltpu.sync_copy(tn_hbm.at[wid], tn_idx)

    def ctx_dma(i, r):
        return pltpu.make_async_copy(w_hbm.at[ctx_idx.at[i]],
                                     bufs[r].at[pl.ds(0, CCH)], sem_cs[r])

    def ot_dma(i, r):
        return pltpu.make_async_copy(ot_hbm.at[tn_idx.at[i]], bufs[r],
                                     sem_os[r])

    doc_dma = pltpu.make_async_copy(d_hbm.at[doc_idx], xbuf, sem_d)
    doc_dma.start()
    for r in range(RING):
        ctx_dma(r, r).start()
    doc_dma.wait()

    def seg_sum(i, buf):
        def sbody(bb, carry):
            b = i * SUB + bb
            for v in range(NLG):
                sl = pl.ds(v * 16, 16)
                xbuf[b, sl] = _tree_sum(
                    [xbuf[b, sl]]
                    + [buf[bb * CTX + j, sl] for j in range(CTX)])
            return carry
        lax.fori_loop(0, SUB, sbody, 0)

    def p1(k, carry):
        i0 = RING * k
        for r in range(RING):
            ctx_dma(i0 + r, r).wait()
            seg_sum(i0 + r, bufs[r])
            ctx_dma(i0 + r + RING, r).start()
        return carry

    lax.fori_loop(0, NIT // RING - 1, p1, 0)
    for r in range(RING):
        i = NIT - RING + r
        ctx_dma(i, r).wait()
        seg_sum(i, bufs[r])
        ot_dma(r, r).start()

    lanes = lax.iota(jnp.int32, 16)
    perms = [lanes ^ sh for sh in (8, 4, 2, 1)]
    masks = [lanes == (t % 16) for t in range(NOISE)]
    zeros16 = jnp.zeros((16,), jnp.float32)

    def dots(i, buf):
        def dbody(bb, carry):
            b = i * SUB + bb
            xv = [xbuf[b, pl.ds(v * 16, 16)] for v in range(NLG)]
            og = [zeros16, zeros16]
            for t in range(NOISE):
                r = bb * NOISE + t
                acc = _tree_sum([xv[v] * buf[r, pl.ds(v * 16, 16)]
                                 for v in range(NLG)])
                for p in perms:
                    acc = acc + _lane_perm(acc, p)
                og[t // 16] = jnp.where(masks[t], acc, og[t // 16])
            outv[bb, pl.ds(0, 16)] = og[0]
            outv[bb, pl.ds(16, 16)] = og[1]
            return carry
        lax.fori_loop(0, SUB, dbody, 0)
        pltpu.sync_copy(outv, out_hbm.at[pl.ds(wid * BPW + i * SUB, SUB)])

    def p2(k, carry):
        i0 = RING * k
        for r in range(RING):
            ot_dma(i0 + r, r).wait()
            dots(i0 + r, bufs[r])
            ot_dma(i0 + r + RING, r).start()
        return carry

    lax.fori_loop(0, NIT // RING - 1, p2, 0)
    for r in range(RING):
        i = NIT - RING + r
        ot_dma(i, r).wait()
        dots(i, bufs[r])


@functools.partial(jax.jit)
def _sc_fwd(doc, ctx, tn, d, w, ot):
    mesh = plsc.VectorSubcoreMesh(core_axis_name="c", subcore_axis_name="s")
    run = pl.kernel(
        _sc_body,
        mesh=mesh,
        out_type=jax.ShapeDtypeStruct((BATCH, OPAD), jnp.float32),
        scratch_types=[
            pltpu.VMEM((BPW,), jnp.int32),
            pltpu.VMEM((NIT, CCH), jnp.int32),
            pltpu.VMEM((NIT, TCH), jnp.int32),
            pltpu.VMEM((BPW, VEC), jnp.float32),
            pltpu.VMEM((TCH, VEC), jnp.float32),
            pltpu.VMEM((TCH, VEC), jnp.float32),
            pltpu.VMEM((TCH, VEC), jnp.float32),
            pltpu.VMEM((TCH, VEC), jnp.float32),
            pltpu.VMEM((SUB, OPAD), jnp.float32),
            pltpu.SemaphoreType.DMA,
            pltpu.SemaphoreType.DMA,
            pltpu.SemaphoreType.DMA,
            pltpu.SemaphoreType.DMA,
            pltpu.SemaphoreType.DMA,
            pltpu.SemaphoreType.DMA,
            pltpu.SemaphoreType.DMA,
            pltpu.SemaphoreType.DMA,
            pltpu.SemaphoreType.DMA,
        ],
    )
    return run(doc, ctx, tn, d, w, ot)


def kernel(context_ids, doc_ids, target_noise_ids, D, W, O):
    ot = jnp.swapaxes(O, 0, 1)
    doc = doc_ids.reshape(NW, BPW)
    ctx = context_ids.reshape(NW, NIT, CCH)
    tn = target_noise_ids.reshape(NW, NIT, TCH)
    out = _sc_fwd(doc, ctx, tn, D, W, ot)
    return out[:, :NOISE]
